# Initial kernel scaffold; baseline (speedup 1.0000x reference)
import jax
import jax.numpy as jnp
from jax.experimental import pallas as pl


def kernel(atom_features, residue_index):
    raise NotImplementedError("write your pallas kernel here")


# speedup vs baseline: 4.9833x; 4.9833x over previous
import functools

import jax
import jax.numpy as jnp
from jax import lax
from jax.experimental import pallas as pl
from jax.experimental.pallas import tpu as pltpu, tpu_sc as plsc

R = 10000
D = 128
RH = R // 2
RPT = 320
RPAD = RPT * 16
RC = 64
B = 128
NC, NS = 2, 16


def _body(n):
    def body(atoms_hbm, idx_hbm, starts_hbm, ends_hbm, out_hbm,
             feat_buf, idx_buf, ones_buf, vbuf, cbuf, svbuf, evbuf,
             accum, counts):
        c = lax.axis_index("c")
        sid = lax.axis_index("s")

        pltpu.sync_copy(starts_hbm.at[c, sid], svbuf)
        pltpu.sync_copy(ends_hbm.at[c, sid], evbuf)
        start = svbuf[...][0]
        end = evbuf[...][0]

        def zrow(i, _):
            for k in range(D // 16):
                vbuf[i, pl.ds(k * 16, 16)] = jnp.zeros((16,), jnp.float32)
            cbuf[i, :] = jnp.zeros((16,), jnp.float32)
            return _
        lax.fori_loop(0, RC, zrow, 0)

        def orow(i, _):
            ones_buf[i, :] = jnp.ones((16,), jnp.float32)
            return _
        lax.fori_loop(0, B, orow, 0)

        rb = sid * RPT
        for j in range(RPT // RC):
            pltpu.sync_copy(vbuf, accum.at[pl.ds(rb + j * RC, RC)])
            pltpu.sync_copy(cbuf, counts.at[pl.ds(rb + j * RC, RC)])
        plsc.subcore_barrier()

        nblk = (end - start + (B - 1)) // B

        def blk(b, _):
            logical = start + b * B
            p0 = pl.multiple_of(jnp.minimum(logical, n - B), 8)
            pltpu.sync_copy(atoms_hbm.at[pl.ds(p0, B)], feat_buf)
            pltpu.sync_copy(idx_hbm.at[pl.ds(p0, B)], idx_buf)
            base = c * RH
            for k in range(B // 16):
                v = idx_buf[pl.ds(k * 16, 16)]
                pos = p0 + k * 16 + lax.iota(jnp.int32, 16)
                local = v - base
                ok = ((pos >= logical) & (pos < end)
                      & (local >= 0) & (local < RH))
                idx_buf[pl.ds(k * 16, 16)] = jnp.where(ok, local, RH)
            pltpu.sync_copy(feat_buf, accum.at[idx_buf], add=True)
            pltpu.sync_copy(ones_buf, counts.at[idx_buf], add=True)
            return _
        lax.fori_loop(0, nblk, blk, 0)
        plsc.subcore_barrier()

        for j in range(RPT // RC):
            pltpu.sync_copy(accum.at[pl.ds(rb + j * RC, RC)], vbuf)
            pltpu.sync_copy(counts.at[pl.ds(rb + j * RC, RC)], cbuf)

            def drow(i, _):
                cnt = jnp.maximum(cbuf[i, :], 1.0)
                for k in range(D // 16):
                    vbuf[i, pl.ds(k * 16, 16)] = (
                        vbuf[i, pl.ds(k * 16, 16)] / cnt)
                return _
            lax.fori_loop(0, RC, drow, 0)
            pltpu.sync_copy(vbuf, out_hbm.at[c, pl.ds(rb + j * RC, RC)])

    return body


@jax.jit
def kernel(atom_features, residue_index):
    n = atom_features.shape[0]

    s = jnp.searchsorted(residue_index, RH, side="left").astype(jnp.int32)
    s8 = (s // 8) * 8
    w = jnp.arange(NS, dtype=jnp.int32)
    start0 = ((w * s) // NS) // 8 * 8
    end0 = jnp.concatenate([start0[1:], s[None]])
    len1 = jnp.int32(n) - s8
    start1 = s8 + ((w * len1) // NS) // 8 * 8
    end1 = jnp.concatenate([start1[1:], jnp.array([n], jnp.int32)])
    starts = jnp.broadcast_to(
        jnp.stack([start0, start1])[:, :, None], (NC, NS, 16)
    ).astype(jnp.int32)
    ends = jnp.broadcast_to(
        jnp.stack([end0, end1])[:, :, None], (NC, NS, 16)
    ).astype(jnp.int32)

    mesh = plsc.VectorSubcoreMesh(core_axis_name="c", subcore_axis_name="s")
    out_pad = pl.kernel(
        _body(n),
        out_type=jax.ShapeDtypeStruct((NC, RPAD, D), jnp.float32),
        mesh=mesh,
        compiler_params=pltpu.CompilerParams(use_tc_tiling_on_sc=False),
        scratch_types=[
            pltpu.VMEM((B, D), jnp.float32),
            pltpu.VMEM((B,), jnp.int32),
            pltpu.VMEM((B, 16), jnp.float32),
            pltpu.VMEM((RC, D), jnp.float32),
            pltpu.VMEM((RC, 16), jnp.float32),
            pltpu.VMEM((16,), jnp.int32),
            pltpu.VMEM((16,), jnp.int32),
            pltpu.VMEM_SHARED((RPAD, D), jnp.float32),
            pltpu.VMEM_SHARED((RPAD, 16), jnp.float32),
        ],
    )(atom_features, residue_index, starts, ends)
    return out_pad[:, :RH, :].reshape(R, D)


# speedup vs baseline: 7.0744x; 1.4196x over previous
import functools

import jax
import jax.numpy as jnp
from jax import lax
from jax.experimental import pallas as pl
from jax.experimental.pallas import tpu as pltpu, tpu_sc as plsc

R = 10000
D = 128
RH = R // 2
RPT = 320
RPAD = RPT * 16
RC = 40
B = 112
NC, NS = 2, 16


def _body(n):
    def body(atoms_hbm, idx_hbm, starts_hbm, ends_hbm, out_hbm,
             featA, featB, idxA, idxB, ones_buf, vbuf, cbuf, svbuf, evbuf,
             semA, semB, accum, counts):
        c = lax.axis_index("c")
        sid = lax.axis_index("s")

        pltpu.sync_copy(starts_hbm.at[c, sid], svbuf)
        pltpu.sync_copy(ends_hbm.at[c, sid], evbuf)
        start = svbuf[...][0]
        end = evbuf[...][0]
        base = c * RH

        def zrow(i, _):
            for k in range(D // 16):
                vbuf[i, pl.ds(k * 16, 16)] = jnp.zeros((16,), jnp.float32)
            cbuf[i, :] = jnp.zeros((16,), jnp.float32)
            return _
        lax.fori_loop(0, RC, zrow, 0)

        def orow(i, _):
            ones_buf[i, :] = jnp.ones((16,), jnp.float32)
            return _
        lax.fori_loop(0, B, orow, 0)

        rb = sid * RPT
        for j in range(RPT // RC):
            pltpu.sync_copy(vbuf, accum.at[pl.ds(rb + j * RC, RC)])
            pltpu.sync_copy(cbuf, counts.at[pl.ds(rb + j * RC, RC)])
        plsc.subcore_barrier()

        nblk = (end - start + (B - 1)) // B
        npair = jnp.maximum(1, (nblk + 1) // 2)

        def p0_of(bb):
            return pl.multiple_of(
                jnp.minimum(start + bb * B, n - B), 8)

        def fire(feat, idx, sem, bb):
            p0 = p0_of(bb)
            pltpu.async_copy(atoms_hbm.at[pl.ds(p0, B)], feat, sem)
            pltpu.async_copy(idx_hbm.at[pl.ds(p0, B)], idx, sem)

        def wait_gather(feat, idx, sem):
            pltpu.make_async_copy(atoms_hbm.at[pl.ds(0, B)], feat, sem).wait()
            pltpu.make_async_copy(idx_hbm.at[pl.ds(0, B)], idx, sem).wait()

        def remap_scat(feat, idx, bb):
            logical = start + bb * B
            p0 = p0_of(bb)
            for k in range(B // 16):
                v = idx[pl.ds(k * 16, 16)]
                pos = p0 + k * 16 + lax.iota(jnp.int32, 16)
                local = v - base
                ok = ((pos >= logical) & (pos < end)
                      & (local >= 0) & (local < RH))
                idx[pl.ds(k * 16, 16)] = jnp.where(ok, local, RH)
            pltpu.sync_copy(feat, accum.at[idx], add=True)
            pltpu.sync_copy(ones_buf, counts.at[idx], add=True)

        fire(featA, idxA, semA, 0)

        def pair(o, _):
            bb = 2 * o
            wait_gather(featA, idxA, semA)
            fire(featB, idxB, semB, bb + 1)
            remap_scat(featA, idxA, bb)
            wait_gather(featB, idxB, semB)
            fire(featA, idxA, semA, bb + 2)
            remap_scat(featB, idxB, bb + 1)
            return _
        lax.fori_loop(0, npair, pair, 0)
        wait_gather(featA, idxA, semA)
        plsc.subcore_barrier()

        for j in range(RPT // RC):
            pltpu.sync_copy(accum.at[pl.ds(rb + j * RC, RC)], vbuf)
            pltpu.sync_copy(counts.at[pl.ds(rb + j * RC, RC)], cbuf)

            def drow(i, _):
                cnt = jnp.maximum(cbuf[i, :], 1.0)
                for k in range(D // 16):
                    vbuf[i, pl.ds(k * 16, 16)] = (
                        vbuf[i, pl.ds(k * 16, 16)] / cnt)
                return _
            lax.fori_loop(0, RC, drow, 0)
            pltpu.sync_copy(vbuf, out_hbm.at[c, pl.ds(rb + j * RC, RC)])

    return body


@jax.jit
def kernel(atom_features, residue_index):
    n = atom_features.shape[0]

    s = jnp.searchsorted(residue_index, RH, side="left").astype(jnp.int32)
    s8 = (s // 8) * 8
    w = jnp.arange(NS, dtype=jnp.int32)
    start0 = ((w * s) // NS) // 8 * 8
    end0 = jnp.concatenate([start0[1:], s[None]])
    len1 = jnp.int32(n) - s8
    start1 = s8 + ((w * len1) // NS) // 8 * 8
    end1 = jnp.concatenate([start1[1:], jnp.array([n], jnp.int32)])
    starts = jnp.broadcast_to(
        jnp.stack([start0, start1])[:, :, None], (NC, NS, 16)
    ).astype(jnp.int32)
    ends = jnp.broadcast_to(
        jnp.stack([end0, end1])[:, :, None], (NC, NS, 16)
    ).astype(jnp.int32)

    mesh = plsc.VectorSubcoreMesh(core_axis_name="c", subcore_axis_name="s")
    out_pad = pl.kernel(
        _body(n),
        out_type=jax.ShapeDtypeStruct((NC, RPAD, D), jnp.float32),
        mesh=mesh,
        compiler_params=pltpu.CompilerParams(use_tc_tiling_on_sc=False),
        scratch_types=[
            pltpu.VMEM((B, D), jnp.float32),
            pltpu.VMEM((B, D), jnp.float32),
            pltpu.VMEM((B,), jnp.int32),
            pltpu.VMEM((B,), jnp.int32),
            pltpu.VMEM((B, 16), jnp.float32),
            pltpu.VMEM((RC, D), jnp.float32),
            pltpu.VMEM((RC, 16), jnp.float32),
            pltpu.VMEM((16,), jnp.int32),
            pltpu.VMEM((16,), jnp.int32),
            pltpu.SemaphoreType.DMA,
            pltpu.SemaphoreType.DMA,
            pltpu.VMEM_SHARED((RPAD, D), jnp.float32),
            pltpu.VMEM_SHARED((RPAD, 16), jnp.float32),
        ],
    )(atom_features, residue_index, starts, ends)
    return out_pad[:, :RH, :].reshape(R, D)
